# initial kernel scaffold (unmeasured)
import jax
import jax.numpy as jnp
from jax import lax
from jax.experimental import pallas as pl
from jax.experimental.pallas import tpu as pltpu


def kernel(
    x,
):
    def body(*refs):
        pass

    out_shape = jax.ShapeDtypeStruct(..., jnp.float32)
    return pl.pallas_call(body, out_shape=out_shape)(...)


# baseline (device time: 315914 ns/iter reference)
import jax
import jax.numpy as jnp
from jax import lax
from jax.experimental import pallas as pl
from jax.experimental.pallas import tpu as pltpu

N_Z = 4


def kernel(x):
    m, n = x.shape
    xb = x.astype(jnp.bfloat16)

    def body(x_ref, out_ref, comm_ref, send_sems, recv_sems):
        my_x = lax.axis_index("x")
        my_y = lax.axis_index("y")
        my_z = lax.axis_index("z")
        up = (my_z + 1) % N_Z
        down = (my_z - 1) % N_Z

        barrier_sem = pltpu.get_barrier_semaphore()
        for nbr in (down, up):
            pl.semaphore_signal(
                barrier_sem,
                inc=1,
                device_id=(my_x, my_y, nbr),
                device_id_type=pl.DeviceIdType.MESH,
            )
        pl.semaphore_wait(barrier_sem, 2)

        out_ref[:, :] = x_ref[:, :].astype(jnp.float32)

        for h in range(N_Z - 1):
            src = x_ref if h == 0 else comm_ref.at[h - 1]
            rdma = pltpu.make_async_remote_copy(
                src_ref=src,
                dst_ref=comm_ref.at[h],
                send_sem=send_sems.at[h],
                recv_sem=recv_sems.at[h],
                device_id=(my_x, my_y, up),
                device_id_type=pl.DeviceIdType.MESH,
            )
            rdma.start()
            rdma.wait()
            out_ref[:, :] += comm_ref[h, :, :].astype(jnp.float32)

    return pl.pallas_call(
        body,
        out_shape=jax.ShapeDtypeStruct((m, n), jnp.float32),
        in_specs=[pl.BlockSpec(memory_space=pltpu.VMEM)],
        out_specs=pl.BlockSpec(memory_space=pltpu.VMEM),
        scratch_shapes=[
            pltpu.VMEM((N_Z - 1, m, n), jnp.bfloat16),
            pltpu.SemaphoreType.DMA((N_Z - 1,)),
            pltpu.SemaphoreType.DMA((N_Z - 1,)),
        ],
        compiler_params=pltpu.CompilerParams(
            collective_id=0,
            vmem_limit_bytes=60 * 1024 * 1024,
        ),
    )(xb)


# device time: 136708 ns/iter; 2.3109x vs baseline; 2.3109x over previous
import functools

import jax
import jax.numpy as jnp
from jax import lax
from jax.experimental import pallas as pl
from jax.experimental.pallas import tpu as pltpu

N_Z = 4
SB = 256
MB = 1024


def kernel(x):
    m, n = x.shape
    assert m == 4 * MB and MB == N_Z * SB
    xb = x.astype(jnp.bfloat16)

    def body(
        x_ref,
        out_ref,
        block_q,
        rs_recv,
        ag_recv,
        blk_x,
        blk_y1,
        blk_y2,
        a_send,
        a_recv,
        b_send,
        b_recv,
        c_sems,
        d1_sems,
        d2_sems,
    ):
        my_x = lax.axis_index("x")
        my_y = lax.axis_index("y")
        my_z = lax.axis_index("z")
        q = 2 * my_x + my_y

        col_peers = [(my_x, my_y, (my_z + o) % N_Z) for o in (1, 2, 3)]
        x_nbr = (1 - my_x, my_y, my_z)
        y_nbr = (my_x, 1 - my_y, my_z)
        partners = col_peers + [x_nbr, y_nbr]

        barrier_sem = pltpu.get_barrier_semaphore()
        for p in partners:
            pl.semaphore_signal(
                barrier_sem, inc=1, device_id=p,
                device_id_type=pl.DeviceIdType.MESH,
            )
        pl.semaphore_wait(barrier_sem, len(partners))

        a_rdmas = []
        for o in (1, 2, 3):
            tz = (my_z + o) % N_Z
            rdma = pltpu.make_async_remote_copy(
                src_ref=x_ref.at[pl.ds(q * MB + tz * SB, SB), :],
                dst_ref=rs_recv.at[o - 1],
                send_sem=a_send.at[o - 1],
                recv_sem=a_recv.at[o - 1],
                device_id=(my_x, my_y, tz),
                device_id_type=pl.DeviceIdType.MESH,
            )
            rdma.start()
            a_rdmas.append(rdma)
        for rdma in a_rdmas:
            rdma.wait_send()
            rdma.wait_recv()

        acc = x_ref[pl.ds(q * MB + my_z * SB, SB), :].astype(jnp.float32)
        for j in range(3):
            acc += rs_recv[j, :, :].astype(jnp.float32)
        block_q[pl.ds(my_z * SB, SB), :] = acc.astype(jnp.bfloat16)

        b_rdmas = []
        for o in (1, 2, 3):
            tz = (my_z + o) % N_Z
            rdma = pltpu.make_async_remote_copy(
                src_ref=block_q.at[pl.ds(my_z * SB, SB), :],
                dst_ref=ag_recv.at[o - 1],
                send_sem=b_send.at[o - 1],
                recv_sem=b_recv.at[o - 1],
                device_id=(my_x, my_y, tz),
                device_id_type=pl.DeviceIdType.MESH,
            )
            rdma.start()
            b_rdmas.append(rdma)
        for rdma in b_rdmas:
            rdma.wait_send()
            rdma.wait_recv()
        for j in range(3):
            src_z = (my_z - (j + 1)) % N_Z
            block_q[pl.ds(src_z * SB, SB), :] = ag_recv[j, :, :]

        rdma_c = pltpu.make_async_remote_copy(
            src_ref=block_q,
            dst_ref=blk_x,
            send_sem=c_sems.at[0],
            recv_sem=c_sems.at[1],
            device_id=x_nbr,
            device_id_type=pl.DeviceIdType.MESH,
        )
        rdma_d1 = pltpu.make_async_remote_copy(
            src_ref=block_q,
            dst_ref=blk_y1,
            send_sem=d1_sems.at[0],
            recv_sem=d1_sems.at[1],
            device_id=y_nbr,
            device_id_type=pl.DeviceIdType.MESH,
        )
        rdma_c.start()
        rdma_d1.start()
        rdma_c.wait_send()
        rdma_c.wait_recv()
        rdma_d1.wait_send()
        rdma_d1.wait_recv()

        rdma_d2 = pltpu.make_async_remote_copy(
            src_ref=blk_x,
            dst_ref=blk_y2,
            send_sem=d2_sems.at[0],
            recv_sem=d2_sems.at[1],
            device_id=y_nbr,
            device_id_type=pl.DeviceIdType.MESH,
        )
        rdma_d2.start()
        rdma_d2.wait_send()
        rdma_d2.wait_recv()

        q_x = 2 * (1 - my_x) + my_y
        q_y = 2 * my_x + (1 - my_y)
        q_d = 2 * (1 - my_x) + (1 - my_y)
        out_ref[pl.ds(q * MB, MB), :] = block_q[:, :].astype(jnp.float32)
        out_ref[pl.ds(q_x * MB, MB), :] = blk_x[:, :].astype(jnp.float32)
        out_ref[pl.ds(q_y * MB, MB), :] = blk_y1[:, :].astype(jnp.float32)
        out_ref[pl.ds(q_d * MB, MB), :] = blk_y2[:, :].astype(jnp.float32)

        @functools.partial(
            pl.run_scoped, second_barrier=pltpu.SemaphoreType.REGULAR
        )
        def _(second_barrier):
            for p in partners:
                pl.semaphore_signal(
                    second_barrier, inc=1, device_id=p,
                    device_id_type=pl.DeviceIdType.MESH,
                )
            pl.semaphore_wait(second_barrier, len(partners))

    return pl.pallas_call(
        body,
        out_shape=jax.ShapeDtypeStruct((m, n), jnp.float32),
        in_specs=[pl.BlockSpec(memory_space=pltpu.VMEM)],
        out_specs=pl.BlockSpec(memory_space=pltpu.VMEM),
        scratch_shapes=[
            pltpu.VMEM((MB, n), jnp.bfloat16),
            pltpu.VMEM((3, SB, n), jnp.bfloat16),
            pltpu.VMEM((3, SB, n), jnp.bfloat16),
            pltpu.VMEM((MB, n), jnp.bfloat16),
            pltpu.VMEM((MB, n), jnp.bfloat16),
            pltpu.VMEM((MB, n), jnp.bfloat16),
            pltpu.SemaphoreType.DMA((3,)),
            pltpu.SemaphoreType.DMA((3,)),
            pltpu.SemaphoreType.DMA((3,)),
            pltpu.SemaphoreType.DMA((3,)),
            pltpu.SemaphoreType.DMA((2,)),
            pltpu.SemaphoreType.DMA((2,)),
            pltpu.SemaphoreType.DMA((2,)),
        ],
        compiler_params=pltpu.CompilerParams(
            collective_id=0,
            vmem_limit_bytes=60 * 1024 * 1024,
        ),
    )(xb)


# device time: 110900 ns/iter; 2.8486x vs baseline; 1.2327x over previous
import functools

import jax
import jax.numpy as jnp
from jax import lax
from jax.experimental import pallas as pl
from jax.experimental.pallas import tpu as pltpu

N_Z = 4
SB = 256
MB = 1024


def kernel(x):
    m, n = x.shape
    assert m == 4 * MB and MB == N_Z * SB
    xb = x.astype(jnp.bfloat16)

    def body(
        x_ref,
        out_ref,
        red,
        rs_recv,
        ag_recv,
        blk_x,
        blk_y1,
        blk_y2,
        a_send,
        a_recv,
        b_send,
        b_recv,
        c_send,
        c_recv,
        d1_send,
        d1_recv,
        d2y_send,
        d2y_recv,
        d2x_send,
        d2x_recv,
    ):
        my_x = lax.axis_index("x")
        my_y = lax.axis_index("y")
        my_z = lax.axis_index("z")
        q = 2 * my_x + my_y
        q_x = 2 * (1 - my_x) + my_y
        q_y = 2 * my_x + (1 - my_y)
        q_d = 2 * (1 - my_x) + (1 - my_y)

        col_peers = [(my_x, my_y, (my_z + o) % N_Z) for o in (1, 2, 3)]
        x_nbr = (1 - my_x, my_y, my_z)
        y_nbr = (my_x, 1 - my_y, my_z)
        partners = col_peers + [x_nbr, y_nbr]

        barrier_sem = pltpu.get_barrier_semaphore()
        for p in partners:
            pl.semaphore_signal(
                barrier_sem, inc=1, device_id=p,
                device_id_type=pl.DeviceIdType.MESH,
            )
        pl.semaphore_wait(barrier_sem, len(partners))

        pending_sends = []

        a_rdmas = []
        for o in (1, 2, 3):
            tz = (my_z + o) % N_Z
            rdma = pltpu.make_async_remote_copy(
                src_ref=x_ref.at[pl.ds(q * MB + tz * SB, SB), :],
                dst_ref=rs_recv.at[o - 1],
                send_sem=a_send.at[o - 1],
                recv_sem=a_recv.at[o - 1],
                device_id=(my_x, my_y, tz),
                device_id_type=pl.DeviceIdType.MESH,
            )
            rdma.start()
            a_rdmas.append(rdma)
        for rdma in a_rdmas:
            rdma.wait_recv()
            pending_sends.append(rdma)

        acc = x_ref[pl.ds(q * MB + my_z * SB, SB), :].astype(jnp.float32)
        for j in range(3):
            acc += rs_recv[j, :, :].astype(jnp.float32)
        red[:, :] = acc.astype(jnp.bfloat16)

        def send_slice(src, dst, s_sem, r_sem, target):
            rdma = pltpu.make_async_remote_copy(
                src_ref=src, dst_ref=dst, send_sem=s_sem, recv_sem=r_sem,
                device_id=target, device_id_type=pl.DeviceIdType.MESH,
            )
            rdma.start()
            pending_sends.append(rdma)
            return rdma

        b_rdmas = []
        for o in (1, 2, 3):
            tz = (my_z + o) % N_Z
            b_rdmas.append(
                send_slice(
                    red, ag_recv.at[o - 1], b_send.at[o - 1],
                    b_recv.at[o - 1], (my_x, my_y, tz),
                )
            )
        send_slice(red, blk_x.at[0], c_send.at[0], c_recv.at[0], x_nbr)
        send_slice(red, blk_y1.at[0], d1_send.at[0], d1_recv.at[0], y_nbr)
        out_ref[pl.ds(q * MB + my_z * SB, SB), :] = red[:, :].astype(
            jnp.float32
        )

        for j in range(3):
            b_rdmas[j].wait_recv()
            src_z = (my_z - (j + 1)) % N_Z
            k = j + 1
            send_slice(
                ag_recv.at[j], blk_x.at[k], c_send.at[k], c_recv.at[k],
                x_nbr,
            )
            send_slice(
                ag_recv.at[j], blk_y1.at[k], d1_send.at[k], d1_recv.at[k],
                y_nbr,
            )
            out_ref[pl.ds(q * MB + src_z * SB, SB), :] = ag_recv[
                j, :, :
            ].astype(jnp.float32)

        c_wait = [
            pltpu.make_async_remote_copy(
                src_ref=red, dst_ref=blk_x.at[k], send_sem=c_send.at[k],
                recv_sem=c_recv.at[k], device_id=x_nbr,
                device_id_type=pl.DeviceIdType.MESH,
            )
            for k in range(4)
        ]
        for k in range(4):
            c_wait[k].wait_recv()
            src_z = (my_z - k) % N_Z
            if k < 2:
                send_slice(
                    blk_x.at[k], blk_y2.at[k], d2y_send.at[k],
                    d2y_recv.at[k], y_nbr,
                )
            out_ref[pl.ds(q_x * MB + src_z * SB, SB), :] = blk_x[
                k, :, :
            ].astype(jnp.float32)

        d1_wait = [
            pltpu.make_async_remote_copy(
                src_ref=red, dst_ref=blk_y1.at[k], send_sem=d1_send.at[k],
                recv_sem=d1_recv.at[k], device_id=y_nbr,
                device_id_type=pl.DeviceIdType.MESH,
            )
            for k in range(4)
        ]
        for k in range(4):
            d1_wait[k].wait_recv()
            src_z = (my_z - k) % N_Z
            if k >= 2:
                send_slice(
                    blk_y1.at[k], blk_y2.at[k], d2x_send.at[k - 2],
                    d2x_recv.at[k - 2], x_nbr,
                )
            out_ref[pl.ds(q_y * MB + src_z * SB, SB), :] = blk_y1[
                k, :, :
            ].astype(jnp.float32)

        d2_wait = [
            pltpu.make_async_remote_copy(
                src_ref=red, dst_ref=blk_y2.at[k],
                send_sem=(d2y_send if k < 2 else d2x_send).at[k % 2],
                recv_sem=(d2y_recv if k < 2 else d2x_recv).at[k % 2],
                device_id=y_nbr if k < 2 else x_nbr,
                device_id_type=pl.DeviceIdType.MESH,
            )
            for k in range(4)
        ]
        for k in range(4):
            d2_wait[k].wait_recv()
            src_z = (my_z - k) % N_Z
            out_ref[pl.ds(q_d * MB + src_z * SB, SB), :] = blk_y2[
                k, :, :
            ].astype(jnp.float32)

        for rdma in pending_sends:
            rdma.wait_send()

        @functools.partial(
            pl.run_scoped, second_barrier=pltpu.SemaphoreType.REGULAR
        )
        def _(second_barrier):
            for p in partners:
                pl.semaphore_signal(
                    second_barrier, inc=1, device_id=p,
                    device_id_type=pl.DeviceIdType.MESH,
                )
            pl.semaphore_wait(second_barrier, len(partners))

    return pl.pallas_call(
        body,
        out_shape=jax.ShapeDtypeStruct((m, n), jnp.float32),
        in_specs=[pl.BlockSpec(memory_space=pltpu.VMEM)],
        out_specs=pl.BlockSpec(memory_space=pltpu.VMEM),
        scratch_shapes=[
            pltpu.VMEM((SB, n), jnp.bfloat16),
            pltpu.VMEM((3, SB, n), jnp.bfloat16),
            pltpu.VMEM((3, SB, n), jnp.bfloat16),
            pltpu.VMEM((4, SB, n), jnp.bfloat16),
            pltpu.VMEM((4, SB, n), jnp.bfloat16),
            pltpu.VMEM((4, SB, n), jnp.bfloat16),
            pltpu.SemaphoreType.DMA((3,)),
            pltpu.SemaphoreType.DMA((3,)),
            pltpu.SemaphoreType.DMA((3,)),
            pltpu.SemaphoreType.DMA((3,)),
            pltpu.SemaphoreType.DMA((4,)),
            pltpu.SemaphoreType.DMA((4,)),
            pltpu.SemaphoreType.DMA((4,)),
            pltpu.SemaphoreType.DMA((4,)),
            pltpu.SemaphoreType.DMA((2,)),
            pltpu.SemaphoreType.DMA((2,)),
            pltpu.SemaphoreType.DMA((2,)),
            pltpu.SemaphoreType.DMA((2,)),
        ],
        compiler_params=pltpu.CompilerParams(
            collective_id=0,
            vmem_limit_bytes=60 * 1024 * 1024,
        ),
    )(xb)


# device time: 104100 ns/iter; 3.0347x vs baseline; 1.0653x over previous
import functools

import jax
import jax.numpy as jnp
from jax import lax
from jax.experimental import pallas as pl
from jax.experimental.pallas import tpu as pltpu

N_Z = 4
SB = 256
MB = 1024


def kernel(x):
    m, n = x.shape
    assert m == 4 * MB and MB == N_Z * SB
    xb = x.astype(jnp.bfloat16)

    def body(
        x_ref,
        out_ref,
        red,
        rs_recv,
        a_send,
        a_recv,
        b_send,
        b_recv,
        c_send,
        c_recv,
        d1_send,
        d1_recv,
        d2y_send,
        d2y_recv,
        d2x_send,
        d2x_recv,
    ):
        my_x = lax.axis_index("x")
        my_y = lax.axis_index("y")
        my_z = lax.axis_index("z")
        q = 2 * my_x + my_y
        q_x = 2 * (1 - my_x) + my_y
        q_y = 2 * my_x + (1 - my_y)
        q_d = 2 * (1 - my_x) + (1 - my_y)

        col_peers = [(my_x, my_y, (my_z + o) % N_Z) for o in (1, 2, 3)]
        x_nbr = (1 - my_x, my_y, my_z)
        y_nbr = (my_x, 1 - my_y, my_z)
        partners = col_peers + [x_nbr, y_nbr]

        barrier_sem = pltpu.get_barrier_semaphore()
        for p in partners:
            pl.semaphore_signal(
                barrier_sem, inc=1, device_id=p,
                device_id_type=pl.DeviceIdType.MESH,
            )
        pl.semaphore_wait(barrier_sem, len(partners))

        pending_sends = []

        def send(src, dst, s_sem, r_sem, target):
            rdma = pltpu.make_async_remote_copy(
                src_ref=src, dst_ref=dst, send_sem=s_sem, recv_sem=r_sem,
                device_id=target, device_id_type=pl.DeviceIdType.MESH,
            )
            rdma.start()
            pending_sends.append(rdma)
            return rdma

        def recv_wait(dst, r_sem):
            pltpu.make_async_remote_copy(
                src_ref=red, dst_ref=dst, send_sem=a_send.at[0],
                recv_sem=r_sem, device_id=x_nbr,
                device_id_type=pl.DeviceIdType.MESH,
            ).wait_recv()

        a_rdmas = []
        for o in (1, 2, 3):
            tz = (my_z + o) % N_Z
            a_rdmas.append(
                send(
                    x_ref.at[pl.ds(q * MB + tz * SB, SB), :],
                    rs_recv.at[o - 1],
                    a_send.at[o - 1],
                    a_recv.at[o - 1],
                    (my_x, my_y, tz),
                )
            )
        for rdma in a_rdmas:
            rdma.wait_recv()

        acc = x_ref[pl.ds(q * MB + my_z * SB, SB), :].astype(jnp.float32)
        for j in range(3):
            acc += rs_recv[j, :, :].astype(jnp.float32)
        red[:, :] = acc.astype(jnp.bfloat16)
        own_rows = q * MB + my_z * SB
        out_ref[pl.ds(own_rows, SB), :] = red[:, :]

        for o in (1, 2, 3):
            send(
                red,
                out_ref.at[pl.ds(own_rows, SB), :],
                b_send.at[o - 1],
                b_recv.at[o - 1],
                (my_x, my_y, (my_z + o) % N_Z),
            )
        own = out_ref.at[pl.ds(own_rows, SB), :]
        send(own, out_ref.at[pl.ds(own_rows, SB), :], c_send.at[0],
             c_recv.at[0], x_nbr)
        send(own, out_ref.at[pl.ds(own_rows, SB), :], d1_send.at[0],
             d1_recv.at[0], y_nbr)

        for j in range(3):
            src_z = (my_z - (j + 1)) % N_Z
            sl = out_ref.at[pl.ds(q * MB + src_z * SB, SB), :]
            recv_wait(sl, b_recv.at[j])
            send(sl, sl, c_send.at[j + 1], c_recv.at[j + 1], x_nbr)
            send(sl, sl, d1_send.at[j + 1], d1_recv.at[j + 1], y_nbr)

        for k in range(4):
            src_z = (my_z - k) % N_Z
            sl = out_ref.at[pl.ds(q_x * MB + src_z * SB, SB), :]
            recv_wait(sl, c_recv.at[k])
            if k < 2:
                send(sl, sl, d2y_send.at[k], d2y_recv.at[k], y_nbr)

        for k in range(4):
            src_z = (my_z - k) % N_Z
            sl = out_ref.at[pl.ds(q_y * MB + src_z * SB, SB), :]
            recv_wait(sl, d1_recv.at[k])
            if k >= 2:
                send(sl, sl, d2x_send.at[k - 2], d2x_recv.at[k - 2], x_nbr)

        for k in range(4):
            src_z = (my_z - k) % N_Z
            sl = out_ref.at[pl.ds(q_d * MB + src_z * SB, SB), :]
            recv_wait(sl, d2y_recv.at[k] if k < 2 else d2x_recv.at[k - 2])

        for rdma in pending_sends:
            rdma.wait_send()

        @functools.partial(
            pl.run_scoped, second_barrier=pltpu.SemaphoreType.REGULAR
        )
        def _(second_barrier):
            for p in partners:
                pl.semaphore_signal(
                    second_barrier, inc=1, device_id=p,
                    device_id_type=pl.DeviceIdType.MESH,
                )
            pl.semaphore_wait(second_barrier, len(partners))

    return pl.pallas_call(
        body,
        out_shape=jax.ShapeDtypeStruct((m, n), jnp.bfloat16),
        in_specs=[pl.BlockSpec(memory_space=pltpu.VMEM)],
        out_specs=pl.BlockSpec(memory_space=pltpu.VMEM),
        scratch_shapes=[
            pltpu.VMEM((SB, n), jnp.bfloat16),
            pltpu.VMEM((3, SB, n), jnp.bfloat16),
            pltpu.SemaphoreType.DMA((3,)),
            pltpu.SemaphoreType.DMA((3,)),
            pltpu.SemaphoreType.DMA((3,)),
            pltpu.SemaphoreType.DMA((3,)),
            pltpu.SemaphoreType.DMA((4,)),
            pltpu.SemaphoreType.DMA((4,)),
            pltpu.SemaphoreType.DMA((4,)),
            pltpu.SemaphoreType.DMA((4,)),
            pltpu.SemaphoreType.DMA((2,)),
            pltpu.SemaphoreType.DMA((2,)),
            pltpu.SemaphoreType.DMA((2,)),
            pltpu.SemaphoreType.DMA((2,)),
        ],
        compiler_params=pltpu.CompilerParams(
            collective_id=0,
            vmem_limit_bytes=60 * 1024 * 1024,
        ),
    )(xb)


# device time: 91895 ns/iter; 3.4378x vs baseline; 1.1328x over previous
import functools

import jax
import jax.numpy as jnp
from jax import lax
from jax.experimental import pallas as pl
from jax.experimental.pallas import tpu as pltpu

N_Z = 4
SB = 256
MB = 1024


def kernel(x):
    m, n = x.shape
    assert m == 4 * MB and MB == N_Z * SB

    def body(
        x_ref,
        out_ref,
        xq,
        xq_b,
        red,
        rs_recv,
        copy_sem,
        a_send,
        a_recv,
        b_send,
        b_recv,
        c_send,
        c_recv,
        d1_send,
        d1_recv,
        d2y_send,
        d2y_recv,
        d2x_send,
        d2x_recv,
    ):
        my_x = lax.axis_index("x")
        my_y = lax.axis_index("y")
        my_z = lax.axis_index("z")
        q = 2 * my_x + my_y
        q_x = 2 * (1 - my_x) + my_y
        q_y = 2 * my_x + (1 - my_y)
        q_d = 2 * (1 - my_x) + (1 - my_y)

        col_peers = [(my_x, my_y, (my_z + o) % N_Z) for o in (1, 2, 3)]
        x_nbr = (1 - my_x, my_y, my_z)
        y_nbr = (my_x, 1 - my_y, my_z)
        partners = col_peers + [x_nbr, y_nbr]

        local_copy = pltpu.make_async_copy(
            x_ref.at[pl.ds(q * MB, MB), :], xq, copy_sem
        )
        local_copy.start()

        barrier_sem = pltpu.get_barrier_semaphore()
        for p in partners:
            pl.semaphore_signal(
                barrier_sem, inc=1, device_id=p,
                device_id_type=pl.DeviceIdType.MESH,
            )
        pl.semaphore_wait(barrier_sem, len(partners))
        local_copy.wait()
        xq_b[:, :] = xq[:, :].astype(jnp.bfloat16)

        pending_sends = []

        def send(src, dst, s_sem, r_sem, target):
            rdma = pltpu.make_async_remote_copy(
                src_ref=src, dst_ref=dst, send_sem=s_sem, recv_sem=r_sem,
                device_id=target, device_id_type=pl.DeviceIdType.MESH,
            )
            rdma.start()
            pending_sends.append(rdma)
            return rdma

        def recv_wait(dst, r_sem):
            pltpu.make_async_remote_copy(
                src_ref=red, dst_ref=dst, send_sem=a_send.at[0],
                recv_sem=r_sem, device_id=x_nbr,
                device_id_type=pl.DeviceIdType.MESH,
            ).wait_recv()

        a_rdmas = []
        for o in (1, 2, 3):
            tz = (my_z + o) % N_Z
            a_rdmas.append(
                send(
                    xq_b.at[pl.ds(tz * SB, SB), :],
                    rs_recv.at[o - 1],
                    a_send.at[o - 1],
                    a_recv.at[o - 1],
                    (my_x, my_y, tz),
                )
            )
        for rdma in a_rdmas:
            rdma.wait_recv()

        acc = xq[pl.ds(my_z * SB, SB), :]
        for j in range(3):
            acc += rs_recv[j, :, :].astype(jnp.float32)
        red[:, :] = acc.astype(jnp.bfloat16)
        own_rows = q * MB + my_z * SB
        out_ref[pl.ds(own_rows, SB), :] = red[:, :]

        for o in (1, 2, 3):
            send(
                red,
                out_ref.at[pl.ds(own_rows, SB), :],
                b_send.at[o - 1],
                b_recv.at[o - 1],
                (my_x, my_y, (my_z + o) % N_Z),
            )
        own = out_ref.at[pl.ds(own_rows, SB), :]
        send(own, out_ref.at[pl.ds(own_rows, SB), :], c_send.at[0],
             c_recv.at[0], x_nbr)
        send(own, out_ref.at[pl.ds(own_rows, SB), :], d1_send.at[0],
             d1_recv.at[0], y_nbr)

        for j in range(3):
            src_z = (my_z - (j + 1)) % N_Z
            sl = out_ref.at[pl.ds(q * MB + src_z * SB, SB), :]
            recv_wait(sl, b_recv.at[j])
            send(sl, sl, c_send.at[j + 1], c_recv.at[j + 1], x_nbr)
            send(sl, sl, d1_send.at[j + 1], d1_recv.at[j + 1], y_nbr)

        for k in range(4):
            src_z = (my_z - k) % N_Z
            sl = out_ref.at[pl.ds(q_x * MB + src_z * SB, SB), :]
            recv_wait(sl, c_recv.at[k])
            if k < 2:
                send(sl, sl, d2y_send.at[k], d2y_recv.at[k], y_nbr)

        for k in range(4):
            src_z = (my_z - k) % N_Z
            sl = out_ref.at[pl.ds(q_y * MB + src_z * SB, SB), :]
            recv_wait(sl, d1_recv.at[k])
            if k >= 2:
                send(sl, sl, d2x_send.at[k - 2], d2x_recv.at[k - 2], x_nbr)

        for k in range(4):
            src_z = (my_z - k) % N_Z
            sl = out_ref.at[pl.ds(q_d * MB + src_z * SB, SB), :]
            recv_wait(sl, d2y_recv.at[k] if k < 2 else d2x_recv.at[k - 2])

        for rdma in pending_sends:
            rdma.wait_send()

        @functools.partial(
            pl.run_scoped, second_barrier=pltpu.SemaphoreType.REGULAR
        )
        def _(second_barrier):
            for p in partners:
                pl.semaphore_signal(
                    second_barrier, inc=1, device_id=p,
                    device_id_type=pl.DeviceIdType.MESH,
                )
            pl.semaphore_wait(second_barrier, len(partners))

    return pl.pallas_call(
        body,
        out_shape=jax.ShapeDtypeStruct((m, n), jnp.bfloat16),
        in_specs=[pl.BlockSpec(memory_space=pl.ANY)],
        out_specs=pl.BlockSpec(memory_space=pltpu.VMEM),
        scratch_shapes=[
            pltpu.VMEM((MB, n), jnp.float32),
            pltpu.VMEM((MB, n), jnp.bfloat16),
            pltpu.VMEM((SB, n), jnp.bfloat16),
            pltpu.VMEM((3, SB, n), jnp.bfloat16),
            pltpu.SemaphoreType.DMA,
            pltpu.SemaphoreType.DMA((3,)),
            pltpu.SemaphoreType.DMA((3,)),
            pltpu.SemaphoreType.DMA((3,)),
            pltpu.SemaphoreType.DMA((3,)),
            pltpu.SemaphoreType.DMA((4,)),
            pltpu.SemaphoreType.DMA((4,)),
            pltpu.SemaphoreType.DMA((4,)),
            pltpu.SemaphoreType.DMA((4,)),
            pltpu.SemaphoreType.DMA((2,)),
            pltpu.SemaphoreType.DMA((2,)),
            pltpu.SemaphoreType.DMA((2,)),
            pltpu.SemaphoreType.DMA((2,)),
        ],
        compiler_params=pltpu.CompilerParams(
            collective_id=0,
            vmem_limit_bytes=60 * 1024 * 1024,
        ),
    )(x)
